# baseline (speedup 1.0000x reference)
import jax
import jax.numpy as jnp
from jax.experimental import pallas as pl
from jax.experimental.pallas import tpu as pltpu

_DIM = 768
_NE = 8
_TOKENS = 32768
_BLK = 4096
_BLK2 = 8192


def _router_body(x_ref, w_ref, b_ref, router_ref):
    logits = jnp.dot(x_ref[...], w_ref[...]) + b_ref[...]
    mx = jnp.max(logits, axis=1, keepdims=True)
    ids = jax.lax.broadcasted_iota(jnp.int32, (_BLK, _NE), 1)
    idx = jnp.min(jnp.where(logits == mx, ids, _NE), axis=1, keepdims=True)
    router_ref[...] = (ids == idx).astype(jnp.float32)


def _idx_body(r_ref, idx_ref):
    ids = jax.lax.broadcasted_iota(jnp.int32, (_BLK2, _NE), 1)
    idxv = jnp.min(jnp.where(r_ref[...] > 0.5, ids, _NE), axis=1, keepdims=True)
    idx_ref[...] = jnp.broadcast_to(idxv, (_BLK2, _NE))


def kernel(x, W, b):
    b2 = b.reshape(1, _NE)
    router = pl.pallas_call(
        _router_body,
        grid=(_TOKENS // _BLK,),
        in_specs=[
            pl.BlockSpec((_BLK, _DIM), lambda i: (i, 0)),
            pl.BlockSpec((_DIM, _NE), lambda i: (0, 0)),
            pl.BlockSpec((1, _NE), lambda i: (0, 0)),
        ],
        out_specs=pl.BlockSpec((_BLK, _NE), lambda i: (i, 0)),
        out_shape=jax.ShapeDtypeStruct((_TOKENS, _NE), jnp.float32),
        compiler_params=pltpu.CompilerParams(
            dimension_semantics=("arbitrary",),
        ),
    )(x, W, b2)
    idx8 = pl.pallas_call(
        _idx_body,
        grid=(_TOKENS // _BLK2,),
        in_specs=[pl.BlockSpec((_BLK2, _NE), lambda i: (i, 0))],
        out_specs=pl.BlockSpec((_BLK2, _NE), lambda i: (i, 0)),
        out_shape=jax.ShapeDtypeStruct((_TOKENS, _NE), jnp.int32),
        compiler_params=pltpu.CompilerParams(
            dimension_semantics=("arbitrary",),
        ),
    )(router)
    return (router, idx8[:, 0:1])


# speedup vs baseline: 1.1555x; 1.1555x over previous
import jax
import jax.numpy as jnp
from jax.experimental import pallas as pl
from jax.experimental.pallas import tpu as pltpu

_DIM = 768
_NE = 8
_TOKENS = 32768
_BLK = 4096


def _router_body(x_ref, w_ref, b_ref, router_ref, idx_ref):
    logits = jnp.dot(x_ref[...], w_ref[...]) + b_ref[...]
    mx = jnp.max(logits, axis=1, keepdims=True)
    ids = jax.lax.broadcasted_iota(jnp.int32, (_BLK, _NE), 1)
    idx = jnp.min(jnp.where(logits == mx, ids, _NE), axis=1, keepdims=True)
    router_ref[...] = (ids == idx).astype(jnp.float32)
    idx_ref[...] = idx


def kernel(x, W, b):
    b2 = b.reshape(1, _NE)
    grid = (_TOKENS // _BLK,)
    router, idx = pl.pallas_call(
        _router_body,
        grid=grid,
        in_specs=[
            pl.BlockSpec((_BLK, _DIM), lambda i: (i, 0)),
            pl.BlockSpec((_DIM, _NE), lambda i: (0, 0)),
            pl.BlockSpec((1, _NE), lambda i: (0, 0)),
        ],
        out_specs=[
            pl.BlockSpec((_BLK, _NE), lambda i: (i, 0)),
            pl.BlockSpec((_BLK, 1), lambda i: (i, 0)),
        ],
        out_shape=[
            jax.ShapeDtypeStruct((_TOKENS, _NE), jnp.float32),
            jax.ShapeDtypeStruct((_TOKENS, 1), jnp.int32),
        ],
        compiler_params=pltpu.CompilerParams(
            dimension_semantics=("arbitrary",),
        ),
    )(x, W, b2)
    return (router, idx)
